# baseline (speedup 1.0000x reference)
import functools

import jax
import jax.numpy as jnp
from jax import lax
from jax.experimental import pallas as pl
from jax.experimental.pallas import tpu as pltpu
from jax.experimental.pallas import tpu_sc as plsc

_B, _H, _W = 64, 48, 48
_MAX_GROUP = 1000
_PADG = 1024
_NROW = _H * _W
_NC, _NS = 2, 16
_RPT = _NROW // _NS
_HALF = 32
_ACC = _HALF * _PADG
_PD = 2
_PH = _H + 2 * _PD


def _leaky(x):
    return jnp.where(x >= 0, x, 0.01 * x)


def _mask_body(fact_r, move_r, transfer_r, pickup_r, dig_r, sd_r, rech_r,
               dn_r, um_ref, fm_ref):
    def orall(ref):
        acc = ref[0]
        for k in range(1, ref.shape[0]):
            acc = jnp.bitwise_or(acc, ref[k])
        return acc

    fm_ref[...] = orall(fact_r)
    um = orall(move_r)
    for r in (transfer_r, pickup_r, dig_r, sd_r, rech_r):
        um = jnp.bitwise_or(um, orall(r))
    um_ref[...] = jnp.bitwise_or(um, dn_r[0])


def _tc_masks(fact, move, transfer, pickup, dig, sd, rech, dn):
    def spec(nch):
        return pl.BlockSpec((nch, 8, _W, _B), lambda h: (0, h, 0, 0))

    out_spec = pl.BlockSpec((8, _W, _B), lambda h: (h, 0, 0))
    return pl.pallas_call(
        _mask_body,
        grid=(_H // 8,),
        in_specs=[spec(4), spec(10), spec(50), spec(10), spec(2), spec(2),
                  spec(2), spec(1)],
        out_specs=[out_spec] * 2,
        out_shape=[jax.ShapeDtypeStruct((_H, _W, _B), jnp.uint8)] * 2,
    )(fact, move, transfer, pickup, dig, sd, rech, dn)


def _dense_body(gf, map_r, fac_r, unit_r, loc_r, um_r, fm_r,
                g_W, g_b, f_W, f_b, u_W, u_b, m_W, m_b, ld_W, ld_b, c_W, c_b,
                cvu_o, cvf_o, idsu_o, idsf_o, scr_me, scr_pq, scr_t):
    zrow = jnp.zeros((2, 1, 50, _B), jnp.float32)
    zcol = jnp.zeros((2, 50, 1, _B), jnp.float32)
    for r in (1, 50):
        scr_me[:, pl.ds(r, 1), pl.ds(1, 50), :] = zrow
        scr_t[pl.ds(r, 1), pl.ds(1, 50), :] = zrow[0]
    for cix in (1, 50):
        scr_me[:, pl.ds(1, 50), pl.ds(cix, 1), :] = zcol
        scr_t[pl.ds(1, 50), pl.ds(cix, 1), :] = zcol[0]
    scr_pq[...] = jnp.zeros((2, _PH, _PH, 128), jnp.float32)

    inner = (pl.ds(_PD, _H), pl.ds(_PD, _W))

    for o in range(2):
        acc = None
        for cix in range(6):
            v = map_r[cix] * m_W[o, cix]
            acc = v if acc is None else acc + v
        scr_me[o, inner[0], inner[1], :] = _leaky(acc + m_b[o])

    for o in range(2):
        acc = None
        for dy in (-1, 0, 1):
            for dx in (-1, 0, 1):
                v = scr_me[o, pl.ds(_PD + dy, _H), pl.ds(_PD + dx, _W), :]
                acc = v if acc is None else acc + v
        q = acc * (1.0 / 9.0)
        scr_pq[o, pl.ds(_PD, _H), pl.ds(_PD, _W), pl.ds(0, _B)] = q
        scr_pq[o, pl.ds(_PD - 1, _H), pl.ds(_PD, _W), pl.ds(_B, _B)] = q

    def conv_pair(hp, _):
        h = hp * 2
        t = None
        for og in range(2):
            z = [None] * 4
            for i in range(2):
                for dy in range(-2, 3):
                    for dx in range(-2, 3):
                        s = scr_pq[i, pl.ds(h + _PD + dy, 1),
                                   pl.ds(_PD + dx, _W), :]
                        for oo in range(4):
                            o = og * 4 + oo
                            w = ld_W[o, i * 25 + (dy + 2) * 5 + (dx + 2)]
                            z[oo] = s * w if z[oo] is None else z[oo] + s * w
            for oo in range(4):
                o = og * 4 + oo
                u = _leaky(z[oo] + ld_b[o])
                v = c_W[0, 8 + o] * u
                t = v if t is None else t + v
        scr_t[pl.ds(h + _PD, 1), pl.ds(_PD, _W), :] = t[:, :, :_B]
        scr_t[pl.ds(h + _PD + 1, 1), pl.ds(_PD, _W), :] = t[:, :, _B:]
        return 0

    lax.fori_loop(0, _H // 2, conv_pair, 0)

    acc = None
    for dy in (-1, 0, 1):
        for dx in (-1, 0, 1):
            v = scr_t[pl.ds(_PD + dy, _H), pl.ds(_PD + dx, _W), :]
            acc = v if acc is None else acc + v
    crit = acc * (1.0 / 9.0)

    g0 = gf[0:1, :]
    g1 = gf[1:2, :]
    sg = c_b[0]
    for o in range(2):
        ge = _leaky(g_W[o, 0] * g0 + g_W[o, 1] * g1 + g_b[o])
        sg = sg + c_W[0, o] * ge
    crit = crit + sg.reshape(1, 1, _B)

    for (ref, wm, bm, nch, base) in ((fac_r, f_W, f_b, 6, 2),
                                     (unit_r, u_W, u_b, 4, 4)):
        for o in range(2):
            acc = None
            for cix in range(nch):
                v = ref[cix] * wm[o, cix]
                acc = v if acc is None else acc + v
            crit = crit + c_W[0, base + o] * _leaky(acc + bm[o])
    for o in range(2):
        crit = crit + c_W[0, 6 + o] * scr_me[o, inner[0], inner[1], :]

    cvu = jnp.where(um_r[...] != 0, crit, 0.0)
    cvf = jnp.where(fm_r[...] != 0, crit, 0.0)
    bl = lax.broadcasted_iota(jnp.int32, (_H, _W, _B), 2)
    boff = (bl % _HALF) * _PADG
    binu = boff + loc_r[1].astype(jnp.int32)
    binf = boff + loc_r[0].astype(jnp.int32)
    zf = jnp.zeros((_H, _W, 128 - _B), jnp.float32)
    zi = jnp.zeros((_H, _W, 128 - _B), jnp.int32)
    cvu_o[...] = jnp.concatenate([cvu, zf], axis=2)
    cvf_o[...] = jnp.concatenate([cvf, zf], axis=2)
    idsu_o[...] = jnp.concatenate([binu, zi], axis=2)
    idsf_o[...] = jnp.concatenate([binf, zi], axis=2)


def _tc_dense(gf, map_f, fac_f, unit_f, loc_f, um, fm,
              g_W, g_b, f_W, f_b, u_W, u_b, m_W, m_b, ld_Wr, ld_b, c_W, c_b):
    vmem = pl.BlockSpec(memory_space=pltpu.VMEM)
    smem = pl.BlockSpec(memory_space=pltpu.SMEM)
    return pl.pallas_call(
        _dense_body,
        in_specs=[vmem] * 7 + [smem] * 12,
        out_specs=[vmem] * 4,
        out_shape=[jax.ShapeDtypeStruct((_H, _W, 128), jnp.float32)] * 2
        + [jax.ShapeDtypeStruct((_H, _W, 128), jnp.int32)] * 2,
        scratch_shapes=[pltpu.VMEM((2, _PH, _PH, _B), jnp.float32),
                        pltpu.VMEM((2, _PH, _PH, 128), jnp.float32),
                        pltpu.VMEM((_PH, _PH, _B), jnp.float32)],
    )(gf, map_f, fac_f, unit_f, loc_f, um, fm,
      g_W, g_b, f_W, f_b, u_W, u_b, m_W, m_b, ld_Wr, ld_b, c_W, c_b)


def _sc_scatter(ids_u, ids_f, cv_u, cv_f):
    mesh = plsc.VectorSubcoreMesh(core_axis_name="c", subcore_axis_name="s")

    @functools.partial(
        pl.kernel,
        out_type=jax.ShapeDtypeStruct((_B * _PADG,), jnp.float32),
        mesh=mesh,
        scratch_types=[
            pltpu.VMEM((_RPT, 128), jnp.int32),
            pltpu.VMEM((_RPT, 128), jnp.float32),
            pltpu.VMEM((_RPT, 128), jnp.int32),
            pltpu.VMEM((_RPT, 128), jnp.float32),
            pltpu.VMEM((2048,), jnp.float32),
            pltpu.VMEM_SHARED((_ACC,), jnp.float32),
            pltpu.SemaphoreType.DMA,
            pltpu.SemaphoreType.DMA,
        ],
        compiler_params=pltpu.CompilerParams(needs_layout_passes=False),
    )
    def run(idsu_hbm, idsf_hbm, cvu_hbm, cvf_hbm, out_hbm,
            fidsu, fcvu, fidsf, fcvf, zbuf, acc_sh, sem, sem2):
        c = lax.axis_index("c")
        s = lax.axis_index("s")

        rows = pl.ds(s * _RPT, _RPT)
        st = [pltpu.async_copy(idsu_hbm.at[rows], fidsu, sem2),
              pltpu.async_copy(cvu_hbm.at[rows], fcvu, sem2),
              pltpu.async_copy(idsf_hbm.at[rows], fidsf, sem2),
              pltpu.async_copy(cvf_hbm.at[rows], fcvf, sem2)]

        zeros16 = jnp.zeros((16,), jnp.float32)

        def zb(i, _):
            zbuf[pl.ds(pl.multiple_of(i * 16, 16), 16)] = zeros16
            return 0

        lax.fori_loop(0, 2048 // 16, zb, 0)
        pltpu.sync_copy(zbuf, acc_sh.at[pl.ds(s * 2048, 2048)])
        for d in st:
            d.wait()

        plsc.subcore_barrier()

        lane0 = pl.multiple_of(c * _HALF, _HALF)
        lanes = pl.ds(lane0, _HALF)

        def scatter_rows(fids, fcv):
            def chunk(i, _):
                base = pl.multiple_of(i * 8, 8)
                ds_ = [pltpu.async_copy(fcv.at[base + jj, lanes],
                                        acc_sh.at[fids.at[base + jj, lanes]],
                                        sem, add=True)
                       for jj in range(8)]
                for d in ds_:
                    d.wait()
                return 0

            lax.fori_loop(0, _RPT // 8, chunk, 0)

        scatter_rows(fidsu, fcvu)
        scatter_rows(fidsf, fcvf)

        plsc.subcore_barrier()

        pltpu.sync_copy(acc_sh.at[pl.ds(s * 2048, 2048)],
                        out_hbm.at[pl.ds(c * _ACC + s * 2048, 2048)])

    return run(ids_u, ids_f, cv_u, cv_f)


def _bm(x):
    perm = tuple(range(1, x.ndim)) + (0,)
    return x.transpose(perm)


def kernel(global_feature, map_feature, factory_feature, unit_feature,
           location_feature, va_factory_act, va_move, va_transfer, va_pickup,
           va_dig, va_self_destruct, va_recharge, va_do_nothing,
           g_W, g_b, f_W, f_b, u_W, u_b, m_W, m_b, ld_W, ld_b, c_W, c_b):
    Bn = global_feature.shape[0]

    def u8(x):
        return _bm(x.astype(jnp.uint8)).reshape(-1, _H, _W, Bn)

    um, fm = _tc_masks(u8(va_factory_act), u8(va_move), u8(va_transfer),
                       u8(va_pickup), u8(va_dig), u8(va_self_destruct),
                       u8(va_recharge), u8(va_do_nothing[:, None]))
    cv_u, cv_f, ids_u, ids_f = _tc_dense(
        global_feature.transpose(1, 0), _bm(map_feature),
        _bm(factory_feature), _bm(unit_feature), _bm(location_feature),
        um, fm,
        g_W, g_b, f_W, f_b, u_W, u_b, m_W, m_b,
        ld_W.reshape(8, 50), ld_b, c_W, c_b)
    out = _sc_scatter(ids_u.reshape(_NROW, 128), ids_f.reshape(_NROW, 128),
                      cv_u.reshape(_NROW, 128), cv_f.reshape(_NROW, 128))
    return out.reshape(Bn, _PADG)[:, :_MAX_GROUP]


# speedup vs baseline: 1.0039x; 1.0039x over previous
import functools

import jax
import jax.numpy as jnp
from jax import lax
from jax.experimental import pallas as pl
from jax.experimental.pallas import tpu as pltpu
from jax.experimental.pallas import tpu_sc as plsc

_B, _H, _W = 64, 48, 48
_MAX_GROUP = 1000
_PADG = 1024
_NROW = _H * _W
_NC, _NS = 2, 16
_RPT = _NROW // _NS
_HALF = 32
_ACC = _HALF * _PADG
_PD = 2
_PH = _H + 2 * _PD


def _leaky(x):
    return jnp.where(x >= 0, x, 0.01 * x)


def _mask_body(fact_r, move_r, transfer_r, pickup_r, dig_r, sd_r, rech_r,
               dn_r, um_ref, fm_ref):
    def orall(ref):
        acc = ref[0]
        for k in range(1, ref.shape[0]):
            acc = jnp.bitwise_or(acc, ref[k])
        return acc

    fm_ref[...] = orall(fact_r)
    um = orall(move_r)
    for r in (transfer_r, pickup_r, dig_r, sd_r, rech_r):
        um = jnp.bitwise_or(um, orall(r))
    um_ref[...] = jnp.bitwise_or(um, dn_r[0])


def _tc_masks(fact, move, transfer, pickup, dig, sd, rech, dn):
    def spec(nch):
        return pl.BlockSpec((nch, 8, _W, _B), lambda h: (0, h, 0, 0))

    out_spec = pl.BlockSpec((8, _W, _B), lambda h: (h, 0, 0))
    return pl.pallas_call(
        _mask_body,
        grid=(_H // 8,),
        in_specs=[spec(4), spec(10), spec(50), spec(10), spec(2), spec(2),
                  spec(2), spec(1)],
        out_specs=[out_spec] * 2,
        out_shape=[jax.ShapeDtypeStruct((_H, _W, _B), jnp.uint8)] * 2,
    )(fact, move, transfer, pickup, dig, sd, rech, dn)


def _dense_body(gf, map_r, fac_r, unit_r, loc_r, um_r, fm_r,
                g_W, g_b, f_W, f_b, u_W, u_b, m_W, m_b, ld_W, ld_b, c_W, c_b,
                cvu_o, cvf_o, idsu_o, idsf_o, scr_me, scr_pq, scr_t):
    zrow = jnp.zeros((2, 1, 50, _B), jnp.float32)
    zcol = jnp.zeros((2, 50, 1, _B), jnp.float32)
    for r in (1, 50):
        scr_me[:, pl.ds(r, 1), pl.ds(1, 50), :] = zrow
        scr_t[pl.ds(r, 1), pl.ds(1, 50), :] = zrow[0]
    for cix in (1, 50):
        scr_me[:, pl.ds(1, 50), pl.ds(cix, 1), :] = zcol
        scr_t[pl.ds(1, 50), pl.ds(cix, 1), :] = zcol[0]
    scr_pq[...] = jnp.zeros((2, _PH, _PH, 128), jnp.float32)

    inner = (pl.ds(_PD, _H), pl.ds(_PD, _W))

    for o in range(2):
        acc = None
        for cix in range(6):
            v = map_r[cix] * m_W[o, cix]
            acc = v if acc is None else acc + v
        scr_me[o, inner[0], inner[1], :] = _leaky(acc + m_b[o])

    for o in range(2):
        acc = None
        for dy in (-1, 0, 1):
            for dx in (-1, 0, 1):
                v = scr_me[o, pl.ds(_PD + dy, _H), pl.ds(_PD + dx, _W), :]
                acc = v if acc is None else acc + v
        q = acc * (1.0 / 9.0)
        scr_pq[o, pl.ds(_PD, _H), pl.ds(_PD, _W), pl.ds(0, _B)] = q
        scr_pq[o, pl.ds(_PD - 1, _H), pl.ds(_PD, _W), pl.ds(_B, _B)] = q

    def conv_pair(hp, _):
        h = hp * 2
        z = [None] * 8
        for i in range(2):
            for dy in range(-2, 3):
                for dx in range(-2, 3):
                    s = scr_pq[i, pl.ds(h + _PD + dy, 1),
                               pl.ds(_PD + dx, _W), :]
                    for o in range(8):
                        w = ld_W[o, i * 25 + (dy + 2) * 5 + (dx + 2)]
                        z[o] = s * w if z[o] is None else z[o] + s * w
        t = None
        for o in range(8):
            u = _leaky(z[o] + ld_b[o])
            v = c_W[0, 8 + o] * u
            t = v if t is None else t + v
        scr_t[pl.ds(h + _PD, 1), pl.ds(_PD, _W), :] = t[:, :, :_B]
        scr_t[pl.ds(h + _PD + 1, 1), pl.ds(_PD, _W), :] = t[:, :, _B:]
        return 0

    lax.fori_loop(0, _H // 2, conv_pair, 0)

    acc = None
    for dy in (-1, 0, 1):
        for dx in (-1, 0, 1):
            v = scr_t[pl.ds(_PD + dy, _H), pl.ds(_PD + dx, _W), :]
            acc = v if acc is None else acc + v
    crit = acc * (1.0 / 9.0)

    g0 = gf[0:1, :]
    g1 = gf[1:2, :]
    sg = c_b[0]
    for o in range(2):
        ge = _leaky(g_W[o, 0] * g0 + g_W[o, 1] * g1 + g_b[o])
        sg = sg + c_W[0, o] * ge
    crit = crit + sg.reshape(1, 1, _B)

    for (ref, wm, bm, nch, base) in ((fac_r, f_W, f_b, 6, 2),
                                     (unit_r, u_W, u_b, 4, 4)):
        for o in range(2):
            acc = None
            for cix in range(nch):
                v = ref[cix] * wm[o, cix]
                acc = v if acc is None else acc + v
            crit = crit + c_W[0, base + o] * _leaky(acc + bm[o])
    for o in range(2):
        crit = crit + c_W[0, 6 + o] * scr_me[o, inner[0], inner[1], :]

    cvu = jnp.where(um_r[...] != 0, crit, 0.0)
    cvf = jnp.where(fm_r[...] != 0, crit, 0.0)
    bl = lax.broadcasted_iota(jnp.int32, (_H, _W, _B), 2)
    boff = (bl % _HALF) * _PADG
    binu = boff + loc_r[1].astype(jnp.int32)
    binf = boff + loc_r[0].astype(jnp.int32)
    zf = jnp.zeros((_H, _W, 128 - _B), jnp.float32)
    zi = jnp.zeros((_H, _W, 128 - _B), jnp.int32)
    cvu_o[...] = jnp.concatenate([cvu, zf], axis=2)
    cvf_o[...] = jnp.concatenate([cvf, zf], axis=2)
    idsu_o[...] = jnp.concatenate([binu, zi], axis=2)
    idsf_o[...] = jnp.concatenate([binf, zi], axis=2)


def _tc_dense(gf, map_f, fac_f, unit_f, loc_f, um, fm,
              g_W, g_b, f_W, f_b, u_W, u_b, m_W, m_b, ld_Wr, ld_b, c_W, c_b):
    vmem = pl.BlockSpec(memory_space=pltpu.VMEM)
    smem = pl.BlockSpec(memory_space=pltpu.SMEM)
    return pl.pallas_call(
        _dense_body,
        in_specs=[vmem] * 7 + [smem] * 12,
        out_specs=[vmem] * 4,
        out_shape=[jax.ShapeDtypeStruct((_H, _W, 128), jnp.float32)] * 2
        + [jax.ShapeDtypeStruct((_H, _W, 128), jnp.int32)] * 2,
        scratch_shapes=[pltpu.VMEM((2, _PH, _PH, _B), jnp.float32),
                        pltpu.VMEM((2, _PH, _PH, 128), jnp.float32),
                        pltpu.VMEM((_PH, _PH, _B), jnp.float32)],
    )(gf, map_f, fac_f, unit_f, loc_f, um, fm,
      g_W, g_b, f_W, f_b, u_W, u_b, m_W, m_b, ld_Wr, ld_b, c_W, c_b)


def _sc_scatter(ids_u, ids_f, cv_u, cv_f):
    mesh = plsc.VectorSubcoreMesh(core_axis_name="c", subcore_axis_name="s")

    @functools.partial(
        pl.kernel,
        out_type=jax.ShapeDtypeStruct((_B * _PADG,), jnp.float32),
        mesh=mesh,
        scratch_types=[
            pltpu.VMEM((_RPT, 128), jnp.int32),
            pltpu.VMEM((_RPT, 128), jnp.float32),
            pltpu.VMEM((_RPT, 128), jnp.int32),
            pltpu.VMEM((_RPT, 128), jnp.float32),
            pltpu.VMEM((2048,), jnp.float32),
            pltpu.VMEM_SHARED((_ACC,), jnp.float32),
            pltpu.SemaphoreType.DMA,
            pltpu.SemaphoreType.DMA,
        ],
        compiler_params=pltpu.CompilerParams(needs_layout_passes=False),
    )
    def run(idsu_hbm, idsf_hbm, cvu_hbm, cvf_hbm, out_hbm,
            fidsu, fcvu, fidsf, fcvf, zbuf, acc_sh, sem, sem2):
        c = lax.axis_index("c")
        s = lax.axis_index("s")

        rows = pl.ds(s * _RPT, _RPT)
        st = [pltpu.async_copy(idsu_hbm.at[rows], fidsu, sem2),
              pltpu.async_copy(cvu_hbm.at[rows], fcvu, sem2),
              pltpu.async_copy(idsf_hbm.at[rows], fidsf, sem2),
              pltpu.async_copy(cvf_hbm.at[rows], fcvf, sem2)]

        zeros16 = jnp.zeros((16,), jnp.float32)

        def zb(i, _):
            zbuf[pl.ds(pl.multiple_of(i * 16, 16), 16)] = zeros16
            return 0

        lax.fori_loop(0, 2048 // 16, zb, 0)
        pltpu.sync_copy(zbuf, acc_sh.at[pl.ds(s * 2048, 2048)])
        for d in st:
            d.wait()

        plsc.subcore_barrier()

        lane0 = pl.multiple_of(c * _HALF, _HALF)
        lanes = pl.ds(lane0, _HALF)

        def scatter_rows(fids, fcv):
            def chunk(i, _):
                base = pl.multiple_of(i * 8, 8)
                ds_ = [pltpu.async_copy(fcv.at[base + jj, lanes],
                                        acc_sh.at[fids.at[base + jj, lanes]],
                                        sem, add=True)
                       for jj in range(8)]
                for d in ds_:
                    d.wait()
                return 0

            lax.fori_loop(0, _RPT // 8, chunk, 0)

        scatter_rows(fidsu, fcvu)
        scatter_rows(fidsf, fcvf)

        plsc.subcore_barrier()

        pltpu.sync_copy(acc_sh.at[pl.ds(s * 2048, 2048)],
                        out_hbm.at[pl.ds(c * _ACC + s * 2048, 2048)])

    return run(ids_u, ids_f, cv_u, cv_f)


def _bm(x):
    perm = tuple(range(1, x.ndim)) + (0,)
    return x.transpose(perm)


def kernel(global_feature, map_feature, factory_feature, unit_feature,
           location_feature, va_factory_act, va_move, va_transfer, va_pickup,
           va_dig, va_self_destruct, va_recharge, va_do_nothing,
           g_W, g_b, f_W, f_b, u_W, u_b, m_W, m_b, ld_W, ld_b, c_W, c_b):
    Bn = global_feature.shape[0]

    def u8(x):
        return _bm(x.astype(jnp.uint8)).reshape(-1, _H, _W, Bn)

    um, fm = _tc_masks(u8(va_factory_act), u8(va_move), u8(va_transfer),
                       u8(va_pickup), u8(va_dig), u8(va_self_destruct),
                       u8(va_recharge), u8(va_do_nothing[:, None]))
    cv_u, cv_f, ids_u, ids_f = _tc_dense(
        global_feature.transpose(1, 0), _bm(map_feature),
        _bm(factory_feature), _bm(unit_feature), _bm(location_feature),
        um, fm,
        g_W, g_b, f_W, f_b, u_W, u_b, m_W, m_b,
        ld_W.reshape(8, 50), ld_b, c_W, c_b)
    out = _sc_scatter(ids_u.reshape(_NROW, 128), ids_f.reshape(_NROW, 128),
                      cv_u.reshape(_NROW, 128), cv_f.reshape(_NROW, 128))
    return out.reshape(Bn, _PADG)[:, :_MAX_GROUP]
